# Initial kernel scaffold; baseline (speedup 1.0000x reference)
import jax
import jax.numpy as jnp
from jax.experimental import pallas as pl


def kernel(x, edge_index, edge_type, W1, root1, b1, W2, root2, b2):
    raise NotImplementedError("write your pallas kernel here")


# speedup vs baseline: 1.2936x; 1.2936x over previous
import functools

import jax
import jax.numpy as jnp
from jax import lax
from jax.experimental import pallas as pl
from jax.experimental.pallas import tpu as pltpu

_B = 8192


def _agg_body(n_nodes, lo, n_loc, x_ref, src_ref, dst_ref, et_ref, acc_ref):
    i = pl.program_id(0)
    trash = n_loc - 8

    @pl.when(i == 0)
    def _init():
        acc_ref[...] = jnp.zeros_like(acc_ref)

    def body(j, c):
        s = src_ref[0, 0, j]
        row = et_ref[0, 0, j] * n_nodes + dst_ref[0, 0, j] - lo
        in_range = jnp.logical_and(row >= 0, row < trash)
        tgt = jnp.where(in_range, row, trash)
        acc_ref[0, pl.ds(tgt, 1), :] += x_ref[0, pl.ds(s, 1), :]
        return c

    lax.fori_loop(0, _B, body, 0, unroll=False)


def _dense1_body(n_rel, acc_ref, cnt_ref, x_ref, w1_ref, root1_ref, b1_ref,
                 out1_ref):
    r = pl.program_id(0)

    @pl.when(r == 0)
    def _init():
        out1_ref[...] = jnp.dot(x_ref[...], root1_ref[...],
                                preferred_element_type=jnp.float32) + \
            b1_ref[...]

    invc = 1.0 / jnp.maximum(cnt_ref[...], 1.0)
    out1_ref[...] += jnp.dot(acc_ref[...] * invc, w1_ref[0],
                             preferred_element_type=jnp.float32)

    @pl.when(r == n_rel - 1)
    def _relu():
        out1_ref[...] = jnp.maximum(out1_ref[...], 0.0)


def _dense2_body(n_rel, acc_ref, cnt_ref, h_ref, w2_ref, root2_ref, b2_ref,
                 out_ref):
    r = pl.program_id(0)

    @pl.when(r == 0)
    def _init():
        out_ref[...] = jnp.dot(h_ref[...], root2_ref[...],
                               preferred_element_type=jnp.float32) + \
            b2_ref[...]

    invc = 1.0 / jnp.maximum(cnt_ref[:, 0:1], 1.0)
    out_ref[...] += jnp.dot(acc_ref[...] * invc, w2_ref[0],
                            preferred_element_type=jnp.float32)

    @pl.when(r == n_rel - 1)
    def _tanh():
        out_ref[...] = jnp.tanh(out_ref[...])


def kernel(x, edge_index, edge_type, W1, root1, b1, W2, root2, b2):
    n_nodes, in_dim = x.shape
    n_rel, _, hid = W1.shape
    out_dim = W2.shape[2]
    n_edges = edge_type.shape[0]

    nb = -(-n_edges // _B)
    pad = nb * _B - n_edges
    src = jnp.concatenate(
        [edge_index[0].astype(jnp.int32), jnp.zeros((pad,), jnp.int32)])
    dst = jnp.concatenate(
        [edge_index[1].astype(jnp.int32), jnp.zeros((pad,), jnp.int32)])
    et = jnp.concatenate(
        [edge_type.astype(jnp.int32), jnp.full((pad,), n_rel, jnp.int32)])
    src = src.reshape(nb, 1, _B)
    dst = dst.reshape(nb, 1, _B)
    et = et.reshape(nb, 1, _B)

    n_tot = n_rel * n_nodes

    cparams = pltpu.CompilerParams(
        dimension_semantics=("arbitrary",),
        vmem_limit_bytes=63 * 1024 * 1024,
    )

    idx_a = pl.BlockSpec((1, 1, _B), lambda i: (i, 0, 0),
                         memory_space=pltpu.SMEM)

    def run_agg(payload, lo, n_loc):
        feat = payload.shape[2]
        return pl.pallas_call(
            functools.partial(_agg_body, n_nodes, lo, n_loc),
            grid=(nb,),
            in_specs=[
                pl.BlockSpec((1, n_nodes, feat), lambda i: (0, 0, 0)),
                idx_a, idx_a, idx_a,
            ],
            out_specs=pl.BlockSpec((1, n_loc, feat), lambda i: (0, 0, 0)),
            out_shape=jax.ShapeDtypeStruct((1, n_loc, feat), jnp.float32),
            compiler_params=cparams,
        )(payload, src, dst, et)

    x3 = x.reshape(1, n_nodes, in_dim)
    half = n_tot // 2
    accf = jnp.concatenate(
        [run_agg(x3, lo, half + 8)[0, :half] for lo in (0, half)], axis=0)

    ones = jnp.ones((1, n_nodes, in_dim), jnp.float32)
    cntf = jnp.concatenate(
        [run_agg(ones, lo, half + 8)[0, :half] for lo in (0, half)], axis=0)

    out1 = pl.pallas_call(
        functools.partial(_dense1_body, n_rel),
        grid=(n_rel,),
        in_specs=[
            pl.BlockSpec((n_nodes, in_dim), lambda r: (r, 0)),
            pl.BlockSpec((n_nodes, in_dim), lambda r: (r, 0)),
            pl.BlockSpec((n_nodes, in_dim), lambda r: (0, 0)),
            pl.BlockSpec((1, in_dim, hid), lambda r: (r, 0, 0)),
            pl.BlockSpec((in_dim, hid), lambda r: (0, 0)),
            pl.BlockSpec((1, hid), lambda r: (0, 0)),
        ],
        out_specs=pl.BlockSpec((n_nodes, hid), lambda r: (0, 0)),
        out_shape=jax.ShapeDtypeStruct((n_nodes, hid), jnp.float32),
        compiler_params=cparams,
    )(accf, cntf, x, W1, root1, b1.reshape(1, hid))

    h3 = out1.reshape(1, n_nodes, hid)
    quart = n_tot // 4
    acc2f = jnp.concatenate(
        [run_agg(h3, lo, quart + 8)[0, :quart]
         for lo in (0, quart, 2 * quart, 3 * quart)], axis=0)

    out = pl.pallas_call(
        functools.partial(_dense2_body, n_rel),
        grid=(n_rel,),
        in_specs=[
            pl.BlockSpec((n_nodes, hid), lambda r: (r, 0)),
            pl.BlockSpec((n_nodes, in_dim), lambda r: (r, 0)),
            pl.BlockSpec((n_nodes, hid), lambda r: (0, 0)),
            pl.BlockSpec((1, hid, out_dim), lambda r: (r, 0, 0)),
            pl.BlockSpec((hid, out_dim), lambda r: (0, 0)),
            pl.BlockSpec((1, out_dim), lambda r: (0, 0)),
        ],
        out_specs=pl.BlockSpec((n_nodes, out_dim), lambda r: (0, 0)),
        out_shape=jax.ShapeDtypeStruct((n_nodes, out_dim), jnp.float32),
        compiler_params=cparams,
    )(acc2f, cntf, out1, W2, root2, b2.reshape(1, out_dim))

    return out


# speedup vs baseline: 2.6288x; 2.0321x over previous
import functools

import jax
import jax.numpy as jnp
from jax import lax
from jax.experimental import pallas as pl
from jax.experimental.pallas import tpu as pltpu

_B = 8192


def _agg_body(n_nodes, lo, n_loc, x_ref, src_ref, dst_ref, et_ref, acc_ref):
    i = pl.program_id(0)
    trash = n_loc - 8

    @pl.when(i == 0)
    def _init():
        acc_ref[...] = jnp.zeros_like(acc_ref)

    def body(j, c):
        s = src_ref[0, 0, j]
        row = et_ref[0, 0, j] * n_nodes + dst_ref[0, 0, j] - lo
        in_range = jnp.logical_and(row >= 0, row < trash)
        tgt = jnp.where(in_range, row, trash)
        acc_ref[0, pl.ds(tgt, 1), :] += x_ref[0, pl.ds(s, 1), :]
        return c

    lax.fori_loop(0, _B, body, 0, unroll=8)


def _dense1_body(n_rel, acc_ref, cnt_ref, x_ref, w1_ref, root1_ref, b1_ref,
                 out1_ref):
    r = pl.program_id(0)

    @pl.when(r == 0)
    def _init():
        out1_ref[...] = jnp.dot(x_ref[...], root1_ref[...],
                                preferred_element_type=jnp.float32) + \
            b1_ref[...]

    invc = 1.0 / jnp.maximum(cnt_ref[...], 1.0)
    out1_ref[...] += jnp.dot(acc_ref[...] * invc, w1_ref[0],
                             preferred_element_type=jnp.float32)

    @pl.when(r == n_rel - 1)
    def _relu():
        out1_ref[...] = jnp.maximum(out1_ref[...], 0.0)


def _dense2_body(n_rel, acc_ref, cnt_ref, h_ref, w2_ref, root2_ref, b2_ref,
                 out_ref):
    r = pl.program_id(0)

    @pl.when(r == 0)
    def _init():
        out_ref[...] = jnp.dot(h_ref[...], root2_ref[...],
                               preferred_element_type=jnp.float32) + \
            b2_ref[...]

    invc = 1.0 / jnp.maximum(cnt_ref[:, 0:1], 1.0)
    out_ref[...] += jnp.dot(acc_ref[...] * invc, w2_ref[0],
                            preferred_element_type=jnp.float32)

    @pl.when(r == n_rel - 1)
    def _tanh():
        out_ref[...] = jnp.tanh(out_ref[...])


def kernel(x, edge_index, edge_type, W1, root1, b1, W2, root2, b2):
    n_nodes, in_dim = x.shape
    n_rel, _, hid = W1.shape
    out_dim = W2.shape[2]
    n_edges = edge_type.shape[0]

    nb = -(-n_edges // _B)
    pad = nb * _B - n_edges
    src = jnp.concatenate(
        [edge_index[0].astype(jnp.int32), jnp.zeros((pad,), jnp.int32)])
    dst = jnp.concatenate(
        [edge_index[1].astype(jnp.int32), jnp.zeros((pad,), jnp.int32)])
    et = jnp.concatenate(
        [edge_type.astype(jnp.int32), jnp.full((pad,), n_rel, jnp.int32)])
    src = src.reshape(nb, 1, _B)
    dst = dst.reshape(nb, 1, _B)
    et = et.reshape(nb, 1, _B)

    n_tot = n_rel * n_nodes

    cparams = pltpu.CompilerParams(
        dimension_semantics=("arbitrary",),
        vmem_limit_bytes=63 * 1024 * 1024,
    )

    idx_a = pl.BlockSpec((1, 1, _B), lambda i: (i, 0, 0),
                         memory_space=pltpu.SMEM)

    def run_agg(payload, lo, n_loc):
        feat = payload.shape[2]
        return pl.pallas_call(
            functools.partial(_agg_body, n_nodes, lo, n_loc),
            grid=(nb,),
            in_specs=[
                pl.BlockSpec((1, n_nodes, feat), lambda i: (0, 0, 0)),
                idx_a, idx_a, idx_a,
            ],
            out_specs=pl.BlockSpec((1, n_loc, feat), lambda i: (0, 0, 0)),
            out_shape=jax.ShapeDtypeStruct((1, n_loc, feat), jnp.float32),
            compiler_params=cparams,
        )(payload, src, dst, et)

    x3 = x.reshape(1, n_nodes, in_dim)
    half = n_tot // 2
    accf = jnp.concatenate(
        [run_agg(x3, lo, half + 8)[0, :half] for lo in (0, half)], axis=0)

    ones = jnp.ones((1, n_nodes, in_dim), jnp.float32)
    cntf = jnp.concatenate(
        [run_agg(ones, lo, half + 8)[0, :half] for lo in (0, half)], axis=0)

    out1 = pl.pallas_call(
        functools.partial(_dense1_body, n_rel),
        grid=(n_rel,),
        in_specs=[
            pl.BlockSpec((n_nodes, in_dim), lambda r: (r, 0)),
            pl.BlockSpec((n_nodes, in_dim), lambda r: (r, 0)),
            pl.BlockSpec((n_nodes, in_dim), lambda r: (0, 0)),
            pl.BlockSpec((1, in_dim, hid), lambda r: (r, 0, 0)),
            pl.BlockSpec((in_dim, hid), lambda r: (0, 0)),
            pl.BlockSpec((1, hid), lambda r: (0, 0)),
        ],
        out_specs=pl.BlockSpec((n_nodes, hid), lambda r: (0, 0)),
        out_shape=jax.ShapeDtypeStruct((n_nodes, hid), jnp.float32),
        compiler_params=cparams,
    )(accf, cntf, x, W1, root1, b1.reshape(1, hid))

    h3 = out1.reshape(1, n_nodes, hid)
    quart = n_tot // 4
    acc2f = jnp.concatenate(
        [run_agg(h3, lo, quart + 8)[0, :quart]
         for lo in (0, quart, 2 * quart, 3 * quart)], axis=0)

    out = pl.pallas_call(
        functools.partial(_dense2_body, n_rel),
        grid=(n_rel,),
        in_specs=[
            pl.BlockSpec((n_nodes, hid), lambda r: (r, 0)),
            pl.BlockSpec((n_nodes, in_dim), lambda r: (r, 0)),
            pl.BlockSpec((n_nodes, hid), lambda r: (0, 0)),
            pl.BlockSpec((1, hid, out_dim), lambda r: (r, 0, 0)),
            pl.BlockSpec((hid, out_dim), lambda r: (0, 0)),
            pl.BlockSpec((1, out_dim), lambda r: (0, 0)),
        ],
        out_specs=pl.BlockSpec((n_nodes, out_dim), lambda r: (0, 0)),
        out_shape=jax.ShapeDtypeStruct((n_nodes, out_dim), jnp.float32),
        compiler_params=cparams,
    )(acc2f, cntf, out1, W2, root2, b2.reshape(1, out_dim))

    return out
